# Initial kernel scaffold; baseline (speedup 1.0000x reference)
import jax
import jax.numpy as jnp
from jax.experimental import pallas as pl


def kernel(z, edge_index, z_table, gin_w1, gin_b1, gin_w2, gin_b2, sage_ws, sage_wn, sage_b, lin1_w, lin1_b, lin2_w, lin2_b):
    raise NotImplementedError("write your pallas kernel here")


# speedup vs baseline: 1.0886x; 1.0886x over previous
import jax
import jax.numpy as jnp
from jax import lax
from jax.experimental import pallas as pl
from jax.experimental.pallas import tpu as pltpu
from jax.experimental.pallas import tpu_sc as plsc

N_NODES = 10000
N_EDGES = 320000
D = 128
MAX_Z = 4000
NC = 2
NS = 16
NW = NC * NS
LANES = 16
HALF = D // NC

NPAD = 10240
NPW = NPAD // NS
EB = 128
EPT = 20480
NB = EPT // EB
NQ = 4
NBQ = NB // NQ
NBUF = 4
DW = 8
EPAD = NS * EPT
NHB = NPW // EB

_mesh = plsc.VectorSubcoreMesh(
    core_axis_name="c", subcore_axis_name="s", num_cores=NC, num_subcores=NS
)


def _edge_body(srcq, dstq, feat, m_hbm, zdeg, agg_out, deg_out, hg_out,
               src_v, dst_v, bufs, ones_v, tab_sh, agg_sh, deg_sh,
               gsem, ssem, dsem):
    c = lax.axis_index("c")
    s = lax.axis_index("s")
    on_c0 = c == 0

    pltpu.sync_copy(m_hbm.at[s], src_v.at[pl.ds(0, NHB)])

    pltpu.sync_copy(
        feat.at[pl.ds(c * NPAD + s * NPW, NPW)],
        tab_sh.at[pl.ds(s * NPW, NPW)],
    )


    zero16 = jnp.zeros((LANES,), jnp.float32)

    @pl.loop(0, EB)
    def _(i):
        for j in range(HALF // LANES):
            bufs[0, i, pl.ds(j * LANES, LANES)] = zero16

    for t in range(NPW // EB):
        pltpu.sync_copy(bufs.at[0], agg_sh.at[pl.ds(s * NPW + t * EB, EB)])

    pltpu.sync_copy(zdeg.at[pl.ds(0, NPW)], deg_sh.at[pl.ds(s * NPW, NPW)])
    pltpu.sync_copy(zdeg.at[pl.ds(NPW, EB)], ones_v)

    plsc.subcore_barrier()

    for t in range(NHB):
        j = t % NBUF
        if t >= NBUF:
            tp = t - NBUF
            pltpu.make_async_copy(
                bufs.at[tp % NBUF],
                hg_out.at[pl.ds(c * NPAD + s * NPW + tp * EB, EB)],
                ssem.at[tp % NBUF],
            ).wait()
        pltpu.async_copy(
            tab_sh.at[src_v.at[t]], bufs.at[j], gsem.at[j]
        )
        pltpu.make_async_copy(
            tab_sh.at[src_v.at[t]], bufs.at[j], gsem.at[j]
        ).wait()
        pltpu.async_copy(
            bufs.at[j],
            hg_out.at[pl.ds(c * NPAD + s * NPW + t * EB, EB)],
            ssem.at[j],
        )
    for t in range(max(0, NHB - NBUF), NHB):
        pltpu.make_async_copy(
            bufs.at[t % NBUF],
            hg_out.at[pl.ds(c * NPAD + s * NPW + t * EB, EB)],
            ssem.at[t % NBUF],
        ).wait()

    def _gather(b, j):
        return pltpu.async_copy(
            tab_sh.at[src_v.at[b]], bufs.at[j], gsem.at[j]
        )

    def _scatter(b, j):
        d = pltpu.async_copy(
            bufs.at[j], agg_sh.at[dst_v.at[b]], ssem.at[j], add=True
        )

        @pl.when(on_c0)
        def _():
            pltpu.async_copy(ones_v, deg_sh.at[dst_v.at[b]], dsem, add=True)

        return d

    for q in range(NQ):
        pltpu.sync_copy(srcq.at[s, q], src_v)
        pltpu.sync_copy(dstq.at[s, q], dst_v)

        for j in range(NBUF):
            _gather(j, j)

        @pl.loop(0, NBQ - NBUF, step=NBUF)
        def _(b0):
            for j in range(NBUF):
                pltpu.make_async_copy(
                    tab_sh.at[src_v.at[b0 + j]], bufs.at[j], gsem.at[j]
                ).wait()
                _scatter(b0 + j, j)
            for j in range(NBUF):
                pltpu.make_async_copy(
                    bufs.at[j], agg_sh.at[dst_v.at[b0 + j]], ssem.at[j]
                ).wait()
                _gather(b0 + NBUF + j, j)

        for j in range(NBUF):
            b = NBQ - NBUF + j
            pltpu.make_async_copy(
                tab_sh.at[src_v.at[b]], bufs.at[j], gsem.at[j]
            ).wait()
            _scatter(b, j)
        for j in range(NBUF):
            b = NBQ - NBUF + j
            pltpu.make_async_copy(
                bufs.at[j], agg_sh.at[dst_v.at[b]], ssem.at[j]
            ).wait()

        @pl.when(on_c0)
        def _():
            @pl.loop(0, NBQ)
            def _(b):
                pltpu.make_async_copy(
                    ones_v, deg_sh.at[dst_v.at[0]], dsem
                ).wait()

    plsc.subcore_barrier()

    pltpu.sync_copy(
        agg_sh.at[pl.ds(s * NPW, NPW)],
        agg_out.at[c, pl.ds(s * NPW, NPW)],
    )

    @pl.when(on_c0)
    def _():
        pltpu.sync_copy(
            deg_sh.at[pl.ds(s * NPW, NPW)],
            deg_out.at[pl.ds(s * NPW, NPW)],
        )


_edge_agg = pl.kernel(
    _edge_body,
    [
        jax.ShapeDtypeStruct((NC, NPAD, HALF), jnp.float32),
        jax.ShapeDtypeStruct((NPAD, DW), jnp.float32),
        jax.ShapeDtypeStruct((NC * NPAD, HALF), jnp.float32),
    ],
    mesh=_mesh,
    scratch_types=[
        pltpu.VMEM((NBQ, EB), jnp.int32),
        pltpu.VMEM((NBQ, EB), jnp.int32),
        pltpu.VMEM((NBUF, EB, HALF), jnp.float32),
        pltpu.VMEM((EB, DW), jnp.float32),
        pltpu.VMEM_SHARED((NPAD, HALF), jnp.float32),
        pltpu.VMEM_SHARED((NPAD, HALF), jnp.float32),
        pltpu.VMEM_SHARED((NPAD, DW), jnp.float32),
        pltpu.SemaphoreType.DMA((NBUF,)),
        pltpu.SemaphoreType.DMA((NBUF,)),
        pltpu.SemaphoreType.DMA,
    ],
    compiler_params=pltpu.CompilerParams(use_tc_tiling_on_sc=False),
    name="edge_agg",
)


RB = 2048
NPB = NPAD // RB


def _gin_body(h0_ref, h1_ref, a0_ref, a1_ref, w1_ref, b1_ref, w2_ref, b2_ref,
              out_ref):
    j = pl.program_id(1)
    pre = jnp.concatenate(
        [h0_ref[...] + a0_ref[...], h1_ref[...] + a1_ref[...]], axis=1
    )
    h1 = jnp.dot(pre, w1_ref[...], preferred_element_type=jnp.float32)
    h1 = jnp.maximum(h1 + b1_ref[...], 0.0)
    h2 = jnp.dot(h1, w2_ref[...], preferred_element_type=jnp.float32)
    h2 = h2 + b2_ref[...]
    out_ref[...] = jnp.where(j == 0, h2[:, :HALF], h2[:, HALF:])


_gin_mlp = pl.pallas_call(
    _gin_body,
    out_shape=jax.ShapeDtypeStruct((NC * NPAD, HALF), jnp.float32),
    grid=(NPB, NC),
    in_specs=[
        pl.BlockSpec((RB, HALF), lambda i, j: (i, 0)),
        pl.BlockSpec((RB, HALF), lambda i, j: (NPB + i, 0)),
        pl.BlockSpec((RB, HALF), lambda i, j: (i, 0)),
        pl.BlockSpec((RB, HALF), lambda i, j: (NPB + i, 0)),
        pl.BlockSpec((D, D), lambda i, j: (0, 0)),
        pl.BlockSpec((1, D), lambda i, j: (0, 0)),
        pl.BlockSpec((D, D), lambda i, j: (0, 0)),
        pl.BlockSpec((1, D), lambda i, j: (0, 0)),
    ],
    out_specs=pl.BlockSpec((RB, HALF), lambda i, j: (j * NPB + i, 0)),
)


def _sage_head_body(h0_ref, h1_ref, n0_ref, n1_ref, d_ref, ws_ref, wn_ref,
                    b_ref, l1_ref, l1b_ref, l2_ref, l2b_ref, out_ref, acc):
    i = pl.program_id(0)
    deg = jnp.sum(d_ref[...], axis=1, keepdims=True)
    nsum = jnp.concatenate([n0_ref[...], n1_ref[...]], axis=1)
    mean = nsum / jnp.maximum(deg, 1.0)
    h2 = jnp.concatenate([h0_ref[...], h1_ref[...]], axis=1)
    h3 = (
        jnp.dot(h2, ws_ref[...], preferred_element_type=jnp.float32)
        + jnp.dot(mean, wn_ref[...], preferred_element_type=jnp.float32)
        + b_ref[...]
    )
    rows = i * RB + lax.broadcasted_iota(jnp.int32, (RB, 1), 0)
    h3 = jnp.where(rows < N_NODES, h3, -jnp.inf)
    bmax = jnp.max(h3, axis=0, keepdims=True)

    @pl.when(i == 0)
    def _():
        acc[pl.ds(0, 1), :] = bmax

    @pl.when(i > 0)
    def _():
        acc[pl.ds(0, 1), :] = jnp.maximum(acc[pl.ds(0, 1), :], bmax)

    @pl.when(i == NPB - 1)
    def _():
        hg = acc[pl.ds(0, 1), :]
        hg = jnp.dot(hg, l1_ref[...], preferred_element_type=jnp.float32)
        hg = jnp.maximum(hg + l1b_ref[...], 0.0)
        out_ref[...] = (
            jnp.dot(hg, l2_ref[...], preferred_element_type=jnp.float32)
            + l2b_ref[...]
        )


_sage_head = pl.pallas_call(
    _sage_head_body,
    out_shape=jax.ShapeDtypeStruct((1, 1), jnp.float32),
    grid=(NPB,),
    in_specs=[
        pl.BlockSpec((RB, HALF), lambda i: (i, 0)),
        pl.BlockSpec((RB, HALF), lambda i: (NPB + i, 0)),
        pl.BlockSpec((RB, HALF), lambda i: (i, 0)),
        pl.BlockSpec((RB, HALF), lambda i: (NPB + i, 0)),
        pl.BlockSpec((RB, DW), lambda i: (i, 0)),
        pl.BlockSpec((D, D), lambda i: (0, 0)),
        pl.BlockSpec((D, D), lambda i: (0, 0)),
        pl.BlockSpec((1, D), lambda i: (0, 0)),
        pl.BlockSpec((D, D), lambda i: (0, 0)),
        pl.BlockSpec((1, D), lambda i: (0, 0)),
        pl.BlockSpec((D, 1), lambda i: (0, 0)),
        pl.BlockSpec((1, 1), lambda i: (0, 0)),
    ],
    out_specs=pl.BlockSpec((1, 1), lambda i: (0, 0)),
    scratch_shapes=[pltpu.VMEM((8, D), jnp.float32)],
)


def kernel(z, edge_index, z_table, gin_w1, gin_b1, gin_w2, gin_b2,
           sage_ws, sage_wn, sage_b, lin1_w, lin1_b, lin2_w, lin2_b):
    z = z.astype(jnp.int32)
    src = edge_index[0].astype(jnp.int32)
    dst = edge_index[1].astype(jnp.int32)

    zp = jnp.concatenate([z, jnp.zeros((NPAD - N_NODES,), jnp.int32)])
    ident = jnp.arange(NPAD, dtype=jnp.int32)
    rpad = ((0, NPAD - MAX_Z), (0, 0))
    tabx = jnp.concatenate([
        jnp.pad(z_table[:, :HALF], rpad), jnp.pad(z_table[:, HALF:], rpad)
    ])

    epad = EPAD - N_EDGES
    srcq = jnp.concatenate([src, jnp.zeros((epad,), jnp.int32)])
    srcq = srcq.reshape(NS, NQ, NBQ, EB)
    dstq = jnp.concatenate([dst, jnp.full((epad,), N_NODES, jnp.int32)])
    dstq = dstq.reshape(NS, NQ, NBQ, EB)
    zsrcq = jnp.take(zp, srcq, axis=0)
    onerow = jnp.where(jnp.arange(DW) == 0, 1.0, 0.0).astype(jnp.float32)
    zdeg = jnp.concatenate(
        [jnp.zeros((NPW, DW), jnp.float32), jnp.tile(onerow, (EB, 1))]
    )

    w1t = gin_w1.T
    b1r = gin_b1.reshape(1, D)
    w2t = gin_w2.T
    b2r = gin_b2.reshape(1, D)

    def scan_body(feat, x):
        m, srcs = x
        agg, deg, hg = _edge_agg(srcs, dstq, feat, m, zdeg)
        aggf = agg.reshape(NC * NPAD, HALF)
        h2 = _gin_mlp(hg, hg, aggf, aggf, w1t, b1r, w2t, b2r)
        return h2, (agg, deg, hg, h2)

    xs_m = jnp.stack([zp, ident]).reshape(2, NS, NHB, EB)
    xs_src = jnp.stack([zsrcq, srcq])
    _, (aggs, degs, _, h2s) = lax.scan(scan_body, tabx, (xs_m, xs_src))

    h2 = h2s[0]
    deg = degs[0]
    nsumf = aggs[1].reshape(NC * NPAD, HALF)
    logits = _sage_head(h2, h2, nsumf, nsumf, deg,
                        sage_ws.T, sage_wn.T, sage_b.reshape(1, D),
                        lin1_w.T, lin1_b.reshape(1, D),
                        lin2_w.T, lin2_b.reshape(1, 1))
    return logits


# speedup vs baseline: 10.1895x; 9.3601x over previous
import jax
import jax.numpy as jnp
from jax import lax
from jax.experimental import pallas as pl
from jax.experimental.pallas import tpu as pltpu
from jax.experimental.pallas import tpu_sc as plsc

N_NODES = 10000
N_EDGES = 320000
D = 128
MAX_Z = 4000
NC = 2
NS = 16
NW = NC * NS
LANES = 16
HALF = D // NC

NPAD = 10240
NPW = NPAD // NS
EBZ = 80
NBZ = NPW // EBZ
EB = 128
EPT = 20480
NB = EPT // EB
NBUF = 4
EPAD = NS * EPT

_mesh = plsc.VectorSubcoreMesh(
    core_axis_name="c", subcore_axis_name="s", num_cores=NC, num_subcores=NS
)


def _embed_body(zq, tabf, h_hbm, zidx_v, rows_v, sem):
    c = lax.axis_index("c")
    s = lax.axis_index("s")
    wid = c * NS + s
    pltpu.sync_copy(zq.at[wid], zidx_v)
    for b in range(NBZ):
        pltpu.async_copy(tabf.at[zidx_v.at[b]], rows_v, sem).wait()
        pltpu.sync_copy(
            rows_v, h_hbm.at[pl.ds(c * NPAD + s * NPW + b * EBZ, EBZ)]
        )


_embed_gather = pl.kernel(
    _embed_body,
    jax.ShapeDtypeStruct((NC * NPAD, HALF), jnp.float32),
    mesh=_mesh,
    scratch_types=[
        pltpu.VMEM((NBZ, EBZ), jnp.int32),
        pltpu.VMEM((EBZ, HALF), jnp.float32),
        pltpu.SemaphoreType.DMA,
    ],
    compiler_params=pltpu.CompilerParams(use_tc_tiling_on_sc=False),
    name="embed_gather",
)


def _make_edge_agg(with_deg):
    out_type = [jax.ShapeDtypeStruct((NC, NPAD, HALF), jnp.float32)]
    scratch = [
        pltpu.VMEM((NB, EB), jnp.int32),
        pltpu.VMEM((NB, EB), jnp.int32),
        pltpu.VMEM((NBUF, EB, HALF), jnp.float32),
        pltpu.VMEM_SHARED((NPAD, HALF), jnp.float32),
        pltpu.SemaphoreType.DMA((NBUF,)),
        pltpu.SemaphoreType.DMA((NBUF,)),
    ]
    if with_deg:
        out_type.append(jax.ShapeDtypeStruct((NPAD, LANES), jnp.float32))
        scratch += [
            pltpu.VMEM((EB, LANES), jnp.float32),
            pltpu.VMEM_SHARED((NPAD, LANES), jnp.float32),
            pltpu.SemaphoreType.DMA,
        ]

    def body(srcq, dstq, feat, agg_out, *rest):
        if with_deg:
            (deg_out, src_v, dst_v, bufs, agg_sh, gsem, ssem,
             ones_v, deg_sh, dsem) = rest
        else:
            src_v, dst_v, bufs, agg_sh, gsem, ssem = rest
        c = lax.axis_index("c")
        s = lax.axis_index("s")
        wid = c * NS + s
        on_c0 = c == 0

        pltpu.sync_copy(srcq.at[wid], src_v)
        pltpu.sync_copy(dstq.at[s], dst_v)

        zero16 = jnp.zeros((LANES,), jnp.float32)

        @pl.loop(0, EB)
        def _(i):
            for j in range(HALF // LANES):
                bufs[0, i, pl.ds(j * LANES, LANES)] = zero16

        for t in range(NPW // EB):
            pltpu.sync_copy(bufs.at[0], agg_sh.at[pl.ds(s * NPW + t * EB, EB)])

        if with_deg:
            @pl.loop(0, EB)
            def _(i):
                ones_v[i, pl.ds(0, LANES)] = zero16

            for t in range(NPW // EB):
                pltpu.sync_copy(ones_v, deg_sh.at[pl.ds(s * NPW + t * EB, EB)])
            onehot = jnp.where(
                lax.iota(jnp.int32, LANES) == 0,
                jnp.float32(1.0),
                jnp.float32(0.0),
            )

            @pl.loop(0, EB)
            def _(i):
                ones_v[i, pl.ds(0, LANES)] = onehot

        plsc.subcore_barrier()

        def _gather(b, j):
            return pltpu.async_copy(
                feat.at[src_v.at[b]], bufs.at[j], gsem.at[j]
            )

        def _scatter(b, j):
            d = pltpu.async_copy(
                bufs.at[j], agg_sh.at[dst_v.at[b]], ssem.at[j], add=True
            )
            if with_deg:
                @pl.when(on_c0)
                def _():
                    pltpu.async_copy(
                        ones_v, deg_sh.at[dst_v.at[b]], dsem, add=True
                    )
            return d

        for j in range(NBUF):
            _gather(j, j)

        @pl.loop(0, NB - NBUF, step=NBUF)
        def _(b0):
            for j in range(NBUF):
                pltpu.make_async_copy(
                    feat.at[src_v.at[b0 + j]], bufs.at[j], gsem.at[j]
                ).wait()
                _scatter(b0 + j, j)
            for j in range(NBUF):
                pltpu.make_async_copy(
                    bufs.at[j], agg_sh.at[dst_v.at[b0 + j]], ssem.at[j]
                ).wait()
                _gather(b0 + NBUF + j, j)

        for j in range(NBUF):
            b = NB - NBUF + j
            pltpu.make_async_copy(
                feat.at[src_v.at[b]], bufs.at[j], gsem.at[j]
            ).wait()
            _scatter(b, j)
        for j in range(NBUF):
            b = NB - NBUF + j
            pltpu.make_async_copy(
                bufs.at[j], agg_sh.at[dst_v.at[b]], ssem.at[j]
            ).wait()

        if with_deg:
            @pl.when(on_c0)
            def _():
                @pl.loop(0, NB)
                def _(b):
                    pltpu.make_async_copy(
                        ones_v, deg_sh.at[dst_v.at[0]], dsem
                    ).wait()

        plsc.subcore_barrier()

        pltpu.sync_copy(
            agg_sh.at[pl.ds(s * NPW, NPW)],
            agg_out.at[c, pl.ds(s * NPW, NPW)],
        )
        if with_deg:
            @pl.when(on_c0)
            def _():
                pltpu.sync_copy(
                    deg_sh.at[pl.ds(s * NPW, NPW)],
                    deg_out.at[pl.ds(s * NPW, NPW)],
                )

    return pl.kernel(
        body, out_type if with_deg else out_type[0],
        mesh=_mesh, scratch_types=scratch,
        compiler_params=pltpu.CompilerParams(use_tc_tiling_on_sc=False),
        name="edge_agg_deg" if with_deg else "edge_agg",
    )


_edge_agg_deg = _make_edge_agg(True)
_edge_agg = _make_edge_agg(False)


RB = 2048
NPB = NPAD // RB


def _gin_body(h0_ref, h1_ref, a0_ref, a1_ref, w1_ref, b1_ref, w2_ref, b2_ref,
              out_ref):
    j = pl.program_id(1)
    pre = jnp.concatenate(
        [h0_ref[...] + a0_ref[...], h1_ref[...] + a1_ref[...]], axis=1
    )
    h1 = jnp.dot(pre, w1_ref[...], preferred_element_type=jnp.float32)
    h1 = jnp.maximum(h1 + b1_ref[...], 0.0)
    h2 = jnp.dot(h1, w2_ref[...], preferred_element_type=jnp.float32)
    h2 = h2 + b2_ref[...]
    out_ref[...] = jnp.where(j == 0, h2[:, :HALF], h2[:, HALF:])


_gin_mlp = pl.pallas_call(
    _gin_body,
    out_shape=jax.ShapeDtypeStruct((NC * NPAD, HALF), jnp.float32),
    grid=(NPB, NC),
    in_specs=[
        pl.BlockSpec((RB, HALF), lambda i, j: (i, 0)),
        pl.BlockSpec((RB, HALF), lambda i, j: (NPB + i, 0)),
        pl.BlockSpec((RB, HALF), lambda i, j: (i, 0)),
        pl.BlockSpec((RB, HALF), lambda i, j: (NPB + i, 0)),
        pl.BlockSpec((D, D), lambda i, j: (0, 0)),
        pl.BlockSpec((1, D), lambda i, j: (0, 0)),
        pl.BlockSpec((D, D), lambda i, j: (0, 0)),
        pl.BlockSpec((1, D), lambda i, j: (0, 0)),
    ],
    out_specs=pl.BlockSpec((RB, HALF), lambda i, j: (j * NPB + i, 0)),
)


def _sage_head_body(h0_ref, h1_ref, n0_ref, n1_ref, d_ref, ws_ref, wn_ref,
                    b_ref, l1_ref, l1b_ref, l2_ref, l2b_ref, out_ref, acc):
    i = pl.program_id(0)
    deg = jnp.sum(d_ref[...], axis=1, keepdims=True)
    nsum = jnp.concatenate([n0_ref[...], n1_ref[...]], axis=1)
    mean = nsum / jnp.maximum(deg, 1.0)
    h2 = jnp.concatenate([h0_ref[...], h1_ref[...]], axis=1)
    h3 = (
        jnp.dot(h2, ws_ref[...], preferred_element_type=jnp.float32)
        + jnp.dot(mean, wn_ref[...], preferred_element_type=jnp.float32)
        + b_ref[...]
    )
    rows = i * RB + lax.broadcasted_iota(jnp.int32, (RB, 1), 0)
    h3 = jnp.where(rows < N_NODES, h3, -jnp.inf)
    bmax = jnp.max(h3, axis=0, keepdims=True)

    @pl.when(i == 0)
    def _():
        acc[pl.ds(0, 1), :] = bmax

    @pl.when(i > 0)
    def _():
        acc[pl.ds(0, 1), :] = jnp.maximum(acc[pl.ds(0, 1), :], bmax)

    @pl.when(i == NPB - 1)
    def _():
        hg = acc[pl.ds(0, 1), :]
        hg = jnp.dot(hg, l1_ref[...], preferred_element_type=jnp.float32)
        hg = jnp.maximum(hg + l1b_ref[...], 0.0)
        out_ref[...] = (
            jnp.dot(hg, l2_ref[...], preferred_element_type=jnp.float32)
            + l2b_ref[...]
        )


_sage_head = pl.pallas_call(
    _sage_head_body,
    out_shape=jax.ShapeDtypeStruct((1, 1), jnp.float32),
    grid=(NPB,),
    in_specs=[
        pl.BlockSpec((RB, HALF), lambda i: (i, 0)),
        pl.BlockSpec((RB, HALF), lambda i: (NPB + i, 0)),
        pl.BlockSpec((RB, HALF), lambda i: (i, 0)),
        pl.BlockSpec((RB, HALF), lambda i: (NPB + i, 0)),
        pl.BlockSpec((RB, LANES), lambda i: (i, 0)),
        pl.BlockSpec((D, D), lambda i: (0, 0)),
        pl.BlockSpec((D, D), lambda i: (0, 0)),
        pl.BlockSpec((1, D), lambda i: (0, 0)),
        pl.BlockSpec((D, D), lambda i: (0, 0)),
        pl.BlockSpec((1, D), lambda i: (0, 0)),
        pl.BlockSpec((D, 1), lambda i: (0, 0)),
        pl.BlockSpec((1, 1), lambda i: (0, 0)),
    ],
    out_specs=pl.BlockSpec((1, 1), lambda i: (0, 0)),
    scratch_shapes=[pltpu.VMEM((8, D), jnp.float32)],
)


def kernel(z, edge_index, z_table, gin_w1, gin_b1, gin_w2, gin_b2,
           sage_ws, sage_wn, sage_b, lin1_w, lin1_b, lin2_w, lin2_b):
    z = z.astype(jnp.int32)
    src = edge_index[0].astype(jnp.int32)
    dst = edge_index[1].astype(jnp.int32)

    zp = jnp.concatenate([z, jnp.zeros((NPAD - N_NODES,), jnp.int32)])
    zq = jnp.stack([zp, zp + MAX_Z]).reshape(NW, NBZ, EBZ)
    tabf = z_table.reshape(MAX_Z, NC, HALF).transpose(1, 0, 2)
    tabf = tabf.reshape(NC * MAX_Z, HALF)

    epad = EPAD - N_EDGES
    spread = jnp.arange(epad, dtype=jnp.int32)
    src_t = jnp.concatenate([src, spread % N_NODES])
    src_t = src_t.reshape(NS, NB, EB)
    srcq = jnp.concatenate([src_t, src_t + NPAD]).reshape(NW, NB, EB)
    dstq = jnp.concatenate([dst, N_NODES + spread % (NPAD - N_NODES)])
    dstq = dstq.reshape(NS, NB, EB)

    h = _embed_gather(zq, tabf)
    agg, deg = _edge_agg_deg(srcq, dstq, h)
    aggf = agg.reshape(NC * NPAD, HALF)
    h2 = _gin_mlp(h, h, aggf, aggf, gin_w1.T, gin_b1.reshape(1, D),
                  gin_w2.T, gin_b2.reshape(1, D))
    nsum = _edge_agg(srcq, dstq, h2)
    nsumf = nsum.reshape(NC * NPAD, HALF)
    logits = _sage_head(h2, h2, nsumf, nsumf, deg,
                        sage_ws.T, sage_wn.T, sage_b.reshape(1, D),
                        lin1_w.T, lin1_b.reshape(1, D),
                        lin2_w.T, lin2_b.reshape(1, 1))
    return logits
